# baseline (speedup 1.0000x reference)
import jax
import jax.numpy as jnp
from jax.experimental import pallas as pl
from jax.experimental.pallas import tpu as pltpu

EPS = 0.001
PROX = 0.15
RB = 256
CB = 256


def _body(posx_r_ref, posy_r_ref, posx_c_ref, posy_c_ref,
          src_c_ref, dst_c_ref, src_r_ref, dst_r_ref, out_ref):
    f32 = jnp.float32
    posx_r = posx_r_ref[...]
    posy_r = posy_r_ref[...]
    posx_c = posx_c_ref[...]
    posy_c = posy_c_ref[...]
    src_c = src_c_ref[...]
    dst_c = dst_c_ref[...]
    src_r = src_r_ref[...]
    dst_r = dst_r_ref[...]
    n = posx_r.shape[1]
    e = src_c.shape[0]

    bf16 = jnp.bfloat16
    i16 = jnp.int16

    def split3(p):
        p1 = p.astype(bf16)
        r1 = p - p1.astype(f32)
        p2 = r1.astype(bf16)
        r2 = r1 - p2.astype(f32)
        p3 = r2.astype(bf16)
        return p1, p2, p3

    p1x, p2x, p3x = split3(posx_c)
    p1y, p2y, p3y = split3(posy_c)
    pieces_col = jnp.concatenate(
        [p1x, p1y, p2x, p2y, p3x, p3y], axis=1)
    lanes16 = jax.lax.broadcasted_iota(i16, (e, n), 1)

    def gather_col(idx_c):
        oh = (lanes16 == idx_c.astype(i16)).astype(bf16)
        g = jnp.dot(oh, pieces_col, preferred_element_type=f32)
        return (g[:, 0:1] + g[:, 2:3] + g[:, 4:5],
                g[:, 1:2] + g[:, 3:4] + g[:, 5:6])

    q1x, q2x, q3x = split3(posx_r)
    q1y, q2y, q3y = split3(posy_r)
    pieces_row = jnp.concatenate(
        [q1x, q1y, q2x, q2y, q3x, q3y], axis=0)
    subl16 = jax.lax.broadcasted_iota(i16, (n, e), 0)

    def gather_row(idx_r):
        oh = (subl16 == idx_r.astype(i16)).astype(bf16)
        g = jnp.dot(pieces_row, oh, preferred_element_type=f32)
        return (g[0:1] + g[2:3] + g[4:5],
                g[1:2] + g[3:4] + g[5:6])

    a1x_c, a1y_c = gather_col(src_c)
    a2x_c, a2y_c = gather_col(dst_c)
    b1x_r, b1y_r = gather_row(src_r)
    b2x_r, b2y_r = gather_row(dst_r)

    d1x_c = a2x_c - a1x_c
    d1y_c = a2y_c - a1y_c
    midx_c = (a1x_c + a2x_c) * 0.5
    midy_c = (a1y_c + a2y_c) * 0.5
    lsq_c = d1x_c * d1x_c + d1y_c * d1y_c
    aa_c = jnp.maximum(lsq_c, 1e-12)
    inv_a_c = 1.0 / aa_c
    hlp_c = jnp.sqrt(jnp.maximum(lsq_c, 1e-24)) * 0.5 + (PROX * 0.5)
    bf16 = jnp.bfloat16
    midx_cb = midx_c.astype(bf16)
    midy_cb = midy_c.astype(bf16)
    hlp_cb = hlp_c.astype(bf16)
    src_cs = src_c.astype(jnp.int16)
    dst_cs = dst_c.astype(jnp.int16)

    d2x_r = b2x_r - b1x_r
    d2y_r = b2y_r - b1y_r
    midx_r = (b1x_r + b2x_r) * 0.5
    midy_r = (b1y_r + b2y_r) * 0.5
    lsq_r = d2x_r * d2x_r + d2y_r * d2y_r
    ee_r = jnp.maximum(lsq_r, 1e-12)
    inv_e_r = 1.0 / ee_r
    hlp_r = jnp.sqrt(jnp.maximum(lsq_r, 1e-24)) * 0.5 + (PROX * 0.5)
    midx_rb = midx_r.astype(bf16)
    midy_rb = midy_r.astype(bf16)
    hlp_rb = hlp_r.astype(bf16)
    src_rs = src_r.astype(jnp.int16)
    dst_rs = dst_r.astype(jnp.int16)

    _L = ((0, 0), (0, 1), (1, 0), (0, 2), (1, 1), (2, 0))

    def lhs12(px, py):
        return jnp.concatenate([px[p] for p, _ in _L] +
                               [py[p] for p, _ in _L], axis=1)

    def rhs12(qx, qy):
        return jnp.concatenate([qx[q] for _, q in _L] +
                               [qy[q] for _, q in _L], axis=0)

    L_b = lhs12(split3(d1x_c), split3(d1y_c))
    L_a = lhs12(split3(a1x_c), split3(a1y_c))
    R_b = rhs12(split3(d2x_r), split3(d2y_r))
    R_d = rhs12(split3(b1x_r), split3(b1y_r))
    c1_c = d1x_c * a1x_c + d1y_c * a1y_c
    f1_r = d2x_r * b1x_r + d2y_r * b1y_r

    DB = 128
    ii = jax.lax.broadcasted_iota(jnp.int16, (DB, DB), 0)
    jj = jax.lax.broadcasted_iota(jnp.int16, (DB, DB), 1)
    tri = jj > ii

    def tile(r0, c0, rb, cb, tri_mask, mats, rl, cl):
        A1x = a1x_c[r0:r0 + rb]
        A1y = a1y_c[r0:r0 + rb]
        D1x = d1x_c[r0:r0 + rb]
        D1y = d1y_c[r0:r0 + rb]
        MIx = midx_cb[r0:r0 + rb]
        MIy = midy_cb[r0:r0 + rb]
        AA = aa_c[r0:r0 + rb]
        IA = inv_a_c[r0:r0 + rb]
        HI = hlp_cb[r0:r0 + rb]
        SI = src_cs[r0:r0 + rb]
        DI = dst_cs[r0:r0 + rb]
        B1x = b1x_r[:, c0:c0 + cb]
        B1y = b1y_r[:, c0:c0 + cb]
        D2x = d2x_r[:, c0:c0 + cb]
        D2y = d2y_r[:, c0:c0 + cb]
        MJx = midx_rb[:, c0:c0 + cb]
        MJy = midy_rb[:, c0:c0 + cb]
        EE = ee_r[:, c0:c0 + cb]
        IE = inv_e_r[:, c0:c0 + cb]
        HJ = hlp_rb[:, c0:c0 + cb]
        SJ = src_rs[:, c0:c0 + cb]
        DJ = dst_rs[:, c0:c0 + cb]

        mb, md, ma = mats
        rx = A1x - B1x
        ry = A1y - B1y
        b = mb[rl:rl + rb, cl:cl + cb]
        c = c1_c[r0:r0 + rb] - md[rl:rl + rb, cl:cl + cb]
        f = ma[rl:rl + rb, cl:cl + cb] - f1_r[:, c0:c0 + cb]
        denom = jnp.maximum(AA * EE - b * b, 1e-12)
        rden = 1.0 / denom
        s = jnp.clip((b * f - c * EE) * rden, 0.0, 1.0)
        t = jnp.clip((b * s + f) * IE, 0.0, 1.0)
        s = jnp.clip((b * t - c) * IA, 0.0, 1.0)
        dx = rx + s * D1x - t * D2x
        dy = ry + s * D1y - t * D2y
        dist = jnp.sqrt(jnp.maximum(dx * dx + dy * dy, 1e-24))

        mdx = MIx - MJx
        mdy = MIy - MJy
        reach = HI + HJ
        prox = (mdx * mdx + mdy * mdy) < (reach * reach)
        share = ((SI == SJ) | (SI == DJ) | (DI == SJ) | (DI == DJ))
        cand = prox & jnp.logical_not(share)
        if tri_mask is not None:
            cand = cand & tri_mask
        candf = cand.astype(bf16).astype(f32)
        contrib = candf * jnp.maximum(EPS - dist, 0.0)
        return jnp.sum(contrib), jnp.sum(candf)

    acc_loss = f32(0.0)
    acc_cnt = f32(0.0)
    nb_r = e // RB
    for bi in range(nb_r):
        r0 = bi * RB
        lb = L_b[r0:r0 + RB]
        la = L_a[r0:r0 + RB]
        mats = (jnp.dot(lb, R_b[:, r0:], preferred_element_type=f32),
                jnp.dot(lb, R_d[:, r0:], preferred_element_type=f32),
                jnp.dot(la, R_b[:, r0:], preferred_element_type=f32))
        for (dr, dc, m) in ((0, 0, tri), (DB, DB, tri), (0, DB, None)):
            dl, dc_ = tile(r0 + dr, r0 + dc, DB, DB, m, mats, dr, dc)
            acc_loss = acc_loss + dl
            acc_cnt = acc_cnt + dc_
        for bj in range(bi + 1, e // CB):
            c0 = bj * CB
            dl, dc_ = tile(r0, c0, RB, CB, None, mats, 0, c0 - r0)
            acc_loss = acc_loss + dl
            acc_cnt = acc_cnt + dc_

    out_ref[0, 0] = acc_loss / jnp.maximum(acc_cnt, 1.0)


def kernel(node_positions, adjacency, edge_index):
    del adjacency
    n = node_positions.shape[1]
    e = edge_index.shape[1]
    pos = node_positions.reshape(n, 2)
    posx = pos[:, 0]
    posy = pos[:, 1]
    src = edge_index[0]
    dst = edge_index[1]
    out = pl.pallas_call(
        _body,
        out_shape=jax.ShapeDtypeStruct((1, 1), jnp.float32),
        out_specs=pl.BlockSpec(memory_space=pltpu.SMEM),
    )(
        posx.reshape(1, n), posy.reshape(1, n),
        posx.reshape(n, 1), posy.reshape(n, 1),
        src.reshape(e, 1), dst.reshape(e, 1),
        src.reshape(1, e), dst.reshape(1, e),
    )
    return out[0, 0]


# speedup vs baseline: 1.0737x; 1.0737x over previous
import jax
import jax.numpy as jnp
from jax.experimental import pallas as pl
from jax.experimental.pallas import tpu as pltpu

EPS = 0.001
PROX = 0.15
RB = 256
CB = 256


def _body(posx_r_ref, posy_r_ref, posx_c_ref, posy_c_ref,
          src_c_ref, dst_c_ref, src_r_ref, dst_r_ref, out_ref):
    f32 = jnp.float32
    posx_r = posx_r_ref[...]
    posy_r = posy_r_ref[...]
    posx_c = posx_c_ref[...]
    posy_c = posy_c_ref[...]
    src_c = src_c_ref[...]
    dst_c = dst_c_ref[...]
    src_r = src_r_ref[...]
    dst_r = dst_r_ref[...]
    n = posx_r.shape[1]
    e = src_c.shape[0]

    bf16 = jnp.bfloat16
    i16 = jnp.int16

    def split3(p):
        p1 = p.astype(bf16)
        r1 = p - p1.astype(f32)
        p2 = r1.astype(bf16)
        r2 = r1 - p2.astype(f32)
        p3 = r2.astype(bf16)
        return p1, p2, p3

    p1x, p2x, p3x = split3(posx_c)
    p1y, p2y, p3y = split3(posy_c)
    pieces_col = jnp.concatenate(
        [p1x, p1y, p2x, p2y, p3x, p3y], axis=1)
    lanes16 = jax.lax.broadcasted_iota(i16, (e, n), 1)

    def gather_col(idx_c):
        oh = (lanes16 == idx_c.astype(i16)).astype(bf16)
        g = jnp.dot(oh, pieces_col, preferred_element_type=f32)
        return (g[:, 0:1] + g[:, 2:3] + g[:, 4:5],
                g[:, 1:2] + g[:, 3:4] + g[:, 5:6])

    q1x, q2x, q3x = split3(posx_r)
    q1y, q2y, q3y = split3(posy_r)
    pieces_row = jnp.concatenate(
        [q1x, q1y, q2x, q2y, q3x, q3y], axis=0)
    subl16 = jax.lax.broadcasted_iota(i16, (n, e), 0)

    def gather_row(idx_r):
        oh = (subl16 == idx_r.astype(i16)).astype(bf16)
        g = jnp.dot(pieces_row, oh, preferred_element_type=f32)
        return (g[0:1] + g[2:3] + g[4:5],
                g[1:2] + g[3:4] + g[5:6])

    a1x_c, a1y_c = gather_col(src_c)
    a2x_c, a2y_c = gather_col(dst_c)
    b1x_r, b1y_r = gather_row(src_r)
    b2x_r, b2y_r = gather_row(dst_r)

    d1x_c = a2x_c - a1x_c
    d1y_c = a2y_c - a1y_c
    midx_c = (a1x_c + a2x_c) * 0.5
    midy_c = (a1y_c + a2y_c) * 0.5
    lsq_c = d1x_c * d1x_c + d1y_c * d1y_c
    aa_c = jnp.maximum(lsq_c, 1e-12)
    inv_a_c = 1.0 / aa_c
    hlp_c = jnp.sqrt(jnp.maximum(lsq_c, 1e-24)) * 0.5 + (PROX * 0.5)
    bf16 = jnp.bfloat16
    midx_cb = midx_c.astype(bf16)
    midy_cb = midy_c.astype(bf16)
    hlp_cb = hlp_c.astype(bf16)
    src_cs = src_c.astype(jnp.int16)
    dst_cs = dst_c.astype(jnp.int16)

    d2x_r = b2x_r - b1x_r
    d2y_r = b2y_r - b1y_r
    midx_r = (b1x_r + b2x_r) * 0.5
    midy_r = (b1y_r + b2y_r) * 0.5
    lsq_r = d2x_r * d2x_r + d2y_r * d2y_r
    ee_r = jnp.maximum(lsq_r, 1e-12)
    inv_e_r = 1.0 / ee_r
    hlp_r = jnp.sqrt(jnp.maximum(lsq_r, 1e-24)) * 0.5 + (PROX * 0.5)
    midx_rb = midx_r.astype(bf16)
    midy_rb = midy_r.astype(bf16)
    hlp_rb = hlp_r.astype(bf16)
    src_rs = src_r.astype(jnp.int16)
    dst_rs = dst_r.astype(jnp.int16)

    DB = 128
    ii = jax.lax.broadcasted_iota(jnp.int16, (DB, DB), 0)
    jj = jax.lax.broadcasted_iota(jnp.int16, (DB, DB), 1)
    tri = jj > ii

    def tile(r0, c0, rb, cb, tri_mask):
        A1x = a1x_c[r0:r0 + rb]
        A1y = a1y_c[r0:r0 + rb]
        D1x = d1x_c[r0:r0 + rb]
        D1y = d1y_c[r0:r0 + rb]
        MIx = midx_cb[r0:r0 + rb]
        MIy = midy_cb[r0:r0 + rb]
        AA = aa_c[r0:r0 + rb]
        IA = inv_a_c[r0:r0 + rb]
        HI = hlp_cb[r0:r0 + rb]
        SI = src_cs[r0:r0 + rb]
        DI = dst_cs[r0:r0 + rb]
        B1x = b1x_r[:, c0:c0 + cb]
        B1y = b1y_r[:, c0:c0 + cb]
        D2x = d2x_r[:, c0:c0 + cb]
        D2y = d2y_r[:, c0:c0 + cb]
        MJx = midx_rb[:, c0:c0 + cb]
        MJy = midy_rb[:, c0:c0 + cb]
        EE = ee_r[:, c0:c0 + cb]
        IE = inv_e_r[:, c0:c0 + cb]
        HJ = hlp_rb[:, c0:c0 + cb]
        SJ = src_rs[:, c0:c0 + cb]
        DJ = dst_rs[:, c0:c0 + cb]

        rx = A1x - B1x
        ry = A1y - B1y
        b = D1x * D2x + D1y * D2y
        c = D1x * rx + D1y * ry
        f = D2x * rx + D2y * ry
        denom = jnp.maximum(AA * EE - b * b, 1e-12)
        rden = 1.0 / denom
        s = jnp.clip((b * f - c * EE) * rden, 0.0, 1.0)
        t = jnp.clip((b * s + f) * IE, 0.0, 1.0)
        s = jnp.clip((b * t - c) * IA, 0.0, 1.0)
        dx = rx + s * D1x - t * D2x
        dy = ry + s * D1y - t * D2y
        dist = jnp.sqrt(jnp.maximum(dx * dx + dy * dy, 1e-24))

        mdx = MIx - MJx
        mdy = MIy - MJy
        reach = HI + HJ
        prox = (mdx * mdx + mdy * mdy) < (reach * reach)
        share = ((SI == SJ) | (SI == DJ) | (DI == SJ) | (DI == DJ))
        cand = prox & jnp.logical_not(share)
        if tri_mask is not None:
            cand = cand & tri_mask
        candf = cand.astype(bf16).astype(f32)
        contrib = candf * jnp.maximum(EPS - dist, 0.0)
        return jnp.sum(contrib), jnp.sum(candf)

    acc_loss = f32(0.0)
    acc_cnt = f32(0.0)
    nb_r = e // RB
    for bi in range(nb_r):
        r0 = bi * RB
        for (dr, dc, m) in ((0, 0, tri), (DB, DB, tri), (0, DB, None)):
            dl, dc_ = tile(r0 + dr, r0 + dc, DB, DB, m)
            acc_loss = acc_loss + dl
            acc_cnt = acc_cnt + dc_
        for bj in range(bi + 1, e // CB):
            dl, dc_ = tile(r0, bj * CB, RB, CB, None)
            acc_loss = acc_loss + dl
            acc_cnt = acc_cnt + dc_

    out_ref[0, 0] = acc_loss / jnp.maximum(acc_cnt, 1.0)


def kernel(node_positions, adjacency, edge_index):
    del adjacency
    n = node_positions.shape[1]
    e = edge_index.shape[1]
    pos = node_positions.reshape(n, 2)
    posx = pos[:, 0]
    posy = pos[:, 1]
    src = edge_index[0]
    dst = edge_index[1]
    out = pl.pallas_call(
        _body,
        out_shape=jax.ShapeDtypeStruct((1, 1), jnp.float32),
        out_specs=pl.BlockSpec(memory_space=pltpu.SMEM),
    )(
        posx.reshape(1, n), posy.reshape(1, n),
        posx.reshape(n, 1), posy.reshape(n, 1),
        src.reshape(e, 1), dst.reshape(e, 1),
        src.reshape(1, e), dst.reshape(1, e),
    )
    return out[0, 0]


# speedup vs baseline: 1.0831x; 1.0088x over previous
import jax
import jax.numpy as jnp
from jax.experimental import pallas as pl
from jax.experimental.pallas import tpu as pltpu

EPS = 0.001
PROX = 0.15
RB = 128
CB = 128


def _body(posx_r_ref, posy_r_ref, posx_c_ref, posy_c_ref,
          src_c_ref, dst_c_ref, src_r_ref, dst_r_ref, out_ref):
    f32 = jnp.float32
    posx_r = posx_r_ref[...]
    posy_r = posy_r_ref[...]
    posx_c = posx_c_ref[...]
    posy_c = posy_c_ref[...]
    src_c = src_c_ref[...]
    dst_c = dst_c_ref[...]
    src_r = src_r_ref[...]
    dst_r = dst_r_ref[...]
    n = posx_r.shape[1]
    e = src_c.shape[0]

    bf16 = jnp.bfloat16
    i16 = jnp.int16

    def split3(p):
        p1 = p.astype(bf16)
        r1 = p - p1.astype(f32)
        p2 = r1.astype(bf16)
        r2 = r1 - p2.astype(f32)
        p3 = r2.astype(bf16)
        return p1, p2, p3

    p1x, p2x, p3x = split3(posx_c)
    p1y, p2y, p3y = split3(posy_c)
    pieces_col = jnp.concatenate(
        [p1x, p1y, p2x, p2y, p3x, p3y], axis=1)
    lanes16 = jax.lax.broadcasted_iota(i16, (e, n), 1)

    def gather_col(idx_c):
        oh = (lanes16 == idx_c.astype(i16)).astype(bf16)
        g = jnp.dot(oh, pieces_col, preferred_element_type=f32)
        return (g[:, 0:1] + g[:, 2:3] + g[:, 4:5],
                g[:, 1:2] + g[:, 3:4] + g[:, 5:6])

    q1x, q2x, q3x = split3(posx_r)
    q1y, q2y, q3y = split3(posy_r)
    pieces_row = jnp.concatenate(
        [q1x, q1y, q2x, q2y, q3x, q3y], axis=0)
    subl16 = jax.lax.broadcasted_iota(i16, (n, e), 0)

    def gather_row(idx_r):
        oh = (subl16 == idx_r.astype(i16)).astype(bf16)
        g = jnp.dot(pieces_row, oh, preferred_element_type=f32)
        return (g[0:1] + g[2:3] + g[4:5],
                g[1:2] + g[3:4] + g[5:6])

    a1x_c, a1y_c = gather_col(src_c)
    a2x_c, a2y_c = gather_col(dst_c)
    b1x_r, b1y_r = gather_row(src_r)
    b2x_r, b2y_r = gather_row(dst_r)

    d1x_c = a2x_c - a1x_c
    d1y_c = a2y_c - a1y_c
    midx_c = (a1x_c + a2x_c) * 0.5
    midy_c = (a1y_c + a2y_c) * 0.5
    lsq_c = d1x_c * d1x_c + d1y_c * d1y_c
    aa_c = jnp.maximum(lsq_c, 1e-12)
    inv_a_c = 1.0 / aa_c
    hlp_c = jnp.sqrt(jnp.maximum(lsq_c, 1e-24)) * 0.5 + (PROX * 0.5)
    bf16 = jnp.bfloat16
    midx_cb = midx_c.astype(bf16)
    midy_cb = midy_c.astype(bf16)
    hlp_cb = hlp_c.astype(bf16)
    src_cs = src_c.astype(jnp.int16)
    dst_cs = dst_c.astype(jnp.int16)

    d2x_r = b2x_r - b1x_r
    d2y_r = b2y_r - b1y_r
    midx_r = (b1x_r + b2x_r) * 0.5
    midy_r = (b1y_r + b2y_r) * 0.5
    lsq_r = d2x_r * d2x_r + d2y_r * d2y_r
    ee_r = jnp.maximum(lsq_r, 1e-12)
    inv_e_r = 1.0 / ee_r
    hlp_r = jnp.sqrt(jnp.maximum(lsq_r, 1e-24)) * 0.5 + (PROX * 0.5)
    midx_rb = midx_r.astype(bf16)
    midy_rb = midy_r.astype(bf16)
    hlp_rb = hlp_r.astype(bf16)
    src_rs = src_r.astype(jnp.int16)
    dst_rs = dst_r.astype(jnp.int16)

    DB = 128
    ii = jax.lax.broadcasted_iota(jnp.int16, (DB, DB), 0)
    jj = jax.lax.broadcasted_iota(jnp.int16, (DB, DB), 1)
    tri = jj > ii

    def tile(r0, c0, rb, cb, tri_mask):
        A1x = a1x_c[r0:r0 + rb]
        A1y = a1y_c[r0:r0 + rb]
        D1x = d1x_c[r0:r0 + rb]
        D1y = d1y_c[r0:r0 + rb]
        MIx = midx_cb[r0:r0 + rb]
        MIy = midy_cb[r0:r0 + rb]
        AA = aa_c[r0:r0 + rb]
        IA = inv_a_c[r0:r0 + rb]
        HI = hlp_cb[r0:r0 + rb]
        SI = src_cs[r0:r0 + rb]
        DI = dst_cs[r0:r0 + rb]
        B1x = b1x_r[:, c0:c0 + cb]
        B1y = b1y_r[:, c0:c0 + cb]
        D2x = d2x_r[:, c0:c0 + cb]
        D2y = d2y_r[:, c0:c0 + cb]
        MJx = midx_rb[:, c0:c0 + cb]
        MJy = midy_rb[:, c0:c0 + cb]
        EE = ee_r[:, c0:c0 + cb]
        IE = inv_e_r[:, c0:c0 + cb]
        HJ = hlp_rb[:, c0:c0 + cb]
        SJ = src_rs[:, c0:c0 + cb]
        DJ = dst_rs[:, c0:c0 + cb]

        rx = A1x - B1x
        ry = A1y - B1y
        b = D1x * D2x + D1y * D2y
        c = D1x * rx + D1y * ry
        f = D2x * rx + D2y * ry
        denom = jnp.maximum(AA * EE - b * b, 1e-12)
        rden = 1.0 / denom
        s = jnp.clip((b * f - c * EE) * rden, 0.0, 1.0)
        t = jnp.clip((b * s + f) * IE, 0.0, 1.0)
        s = jnp.clip((b * t - c) * IA, 0.0, 1.0)
        dx = rx + s * D1x - t * D2x
        dy = ry + s * D1y - t * D2y
        dist = jnp.sqrt(jnp.maximum(dx * dx + dy * dy, 1e-24))

        mdx = MIx - MJx
        mdy = MIy - MJy
        reach = HI + HJ
        prox = (mdx * mdx + mdy * mdy) < (reach * reach)
        share = ((SI == SJ) | (SI == DJ) | (DI == SJ) | (DI == DJ))
        cand = prox & jnp.logical_not(share)
        if tri_mask is not None:
            cand = cand & tri_mask
        candf = cand.astype(bf16).astype(f32)
        contrib = candf * jnp.maximum(EPS - dist, 0.0)
        return jnp.sum(contrib), jnp.sum(candf)

    acc_loss = f32(0.0)
    acc_cnt = f32(0.0)
    nb_r = e // RB
    for bi in range(nb_r):
        r0 = bi * RB
        dl, dc_ = tile(r0, r0, DB, DB, tri)
        acc_loss = acc_loss + dl
        acc_cnt = acc_cnt + dc_
        for bj in range(bi + 1, e // CB):
            dl, dc_ = tile(r0, bj * CB, RB, CB, None)
            acc_loss = acc_loss + dl
            acc_cnt = acc_cnt + dc_

    out_ref[0, 0] = acc_loss / jnp.maximum(acc_cnt, 1.0)


def kernel(node_positions, adjacency, edge_index):
    del adjacency
    n = node_positions.shape[1]
    e = edge_index.shape[1]
    pos = node_positions.reshape(n, 2)
    posx = pos[:, 0]
    posy = pos[:, 1]
    src = edge_index[0]
    dst = edge_index[1]
    out = pl.pallas_call(
        _body,
        out_shape=jax.ShapeDtypeStruct((1, 1), jnp.float32),
        out_specs=pl.BlockSpec(memory_space=pltpu.SMEM),
    )(
        posx.reshape(1, n), posy.reshape(1, n),
        posx.reshape(n, 1), posy.reshape(n, 1),
        src.reshape(e, 1), dst.reshape(e, 1),
        src.reshape(1, e), dst.reshape(1, e),
    )
    return out[0, 0]


# speedup vs baseline: 1.0833x; 1.0002x over previous
import jax
import jax.numpy as jnp
from jax.experimental import pallas as pl
from jax.experimental.pallas import tpu as pltpu

EPS = 0.001
PROX = 0.15
RB = 128
CB = 128


def _body(posx_r_ref, posy_r_ref, posx_c_ref, posy_c_ref,
          src_c_ref, dst_c_ref, src_r_ref, dst_r_ref, out_ref):
    f32 = jnp.float32
    posx_r = posx_r_ref[...]
    posy_r = posy_r_ref[...]
    posx_c = posx_c_ref[...]
    posy_c = posy_c_ref[...]
    src_c = src_c_ref[...]
    dst_c = dst_c_ref[...]
    src_r = src_r_ref[...]
    dst_r = dst_r_ref[...]
    n = posx_r.shape[1]
    e = src_c.shape[0]

    bf16 = jnp.bfloat16
    i16 = jnp.int16

    def split3(p):
        p1 = p.astype(bf16)
        r1 = p - p1.astype(f32)
        p2 = r1.astype(bf16)
        r2 = r1 - p2.astype(f32)
        p3 = r2.astype(bf16)
        return p1, p2, p3

    p1x, p2x, p3x = split3(posx_c)
    p1y, p2y, p3y = split3(posy_c)
    pieces_col = jnp.concatenate(
        [p1x, p1y, p2x, p2y, p3x, p3y], axis=1)
    lanes16 = jax.lax.broadcasted_iota(i16, (e, n), 1)

    def gather_col(idx_c):
        oh = (lanes16 == idx_c.astype(i16)).astype(bf16)
        g = jnp.dot(oh, pieces_col, preferred_element_type=f32)
        return (g[:, 0:1] + g[:, 2:3] + g[:, 4:5],
                g[:, 1:2] + g[:, 3:4] + g[:, 5:6])

    q1x, q2x, q3x = split3(posx_r)
    q1y, q2y, q3y = split3(posy_r)
    pieces_row = jnp.concatenate(
        [q1x, q1y, q2x, q2y, q3x, q3y], axis=0)
    subl16 = jax.lax.broadcasted_iota(i16, (n, e), 0)

    def gather_row(idx_r):
        oh = (subl16 == idx_r.astype(i16)).astype(bf16)
        g = jnp.dot(pieces_row, oh, preferred_element_type=f32)
        return (g[0:1] + g[2:3] + g[4:5],
                g[1:2] + g[3:4] + g[5:6])

    a1x_c, a1y_c = gather_col(src_c)
    a2x_c, a2y_c = gather_col(dst_c)
    b1x_r, b1y_r = gather_row(src_r)
    b2x_r, b2y_r = gather_row(dst_r)

    d1x_c = a2x_c - a1x_c
    d1y_c = a2y_c - a1y_c
    midx_c = (a1x_c + a2x_c) * 0.5
    midy_c = (a1y_c + a2y_c) * 0.5
    lsq_c = d1x_c * d1x_c + d1y_c * d1y_c
    aa_c = jnp.maximum(lsq_c, 1e-12)
    inv_a_c = 1.0 / aa_c
    hlp_c = jnp.sqrt(jnp.maximum(lsq_c, 1e-24)) * 0.5 + (PROX * 0.5)
    bf16 = jnp.bfloat16
    midx_cb = midx_c.astype(bf16)
    midy_cb = midy_c.astype(bf16)
    hlp_cb = hlp_c.astype(bf16)
    src_cs = src_c.astype(jnp.int16)
    dst_cs = dst_c.astype(jnp.int16)

    d2x_r = b2x_r - b1x_r
    d2y_r = b2y_r - b1y_r
    midx_r = (b1x_r + b2x_r) * 0.5
    midy_r = (b1y_r + b2y_r) * 0.5
    lsq_r = d2x_r * d2x_r + d2y_r * d2y_r
    ee_r = jnp.maximum(lsq_r, 1e-12)
    inv_e_r = 1.0 / ee_r
    hlp_r = jnp.sqrt(jnp.maximum(lsq_r, 1e-24)) * 0.5 + (PROX * 0.5)
    midx_rb = midx_r.astype(bf16)
    midy_rb = midy_r.astype(bf16)
    hlp_rb = hlp_r.astype(bf16)
    src_rs = src_r.astype(jnp.int16)
    dst_rs = dst_r.astype(jnp.int16)

    DB = 128
    ii = jax.lax.broadcasted_iota(jnp.int16, (DB, DB), 0)
    jj = jax.lax.broadcasted_iota(jnp.int16, (DB, DB), 1)
    tri = jj > ii

    def tile(r0, c0, rb, cb, tri_mask):
        A1x = a1x_c[r0:r0 + rb]
        A1y = a1y_c[r0:r0 + rb]
        D1x = d1x_c[r0:r0 + rb]
        D1y = d1y_c[r0:r0 + rb]
        MIx = midx_cb[r0:r0 + rb]
        MIy = midy_cb[r0:r0 + rb]
        AA = aa_c[r0:r0 + rb]
        IA = inv_a_c[r0:r0 + rb]
        HI = hlp_cb[r0:r0 + rb]
        SI = src_cs[r0:r0 + rb]
        DI = dst_cs[r0:r0 + rb]
        B1x = b1x_r[:, c0:c0 + cb]
        B1y = b1y_r[:, c0:c0 + cb]
        D2x = d2x_r[:, c0:c0 + cb]
        D2y = d2y_r[:, c0:c0 + cb]
        MJx = midx_rb[:, c0:c0 + cb]
        MJy = midy_rb[:, c0:c0 + cb]
        EE = ee_r[:, c0:c0 + cb]
        IE = inv_e_r[:, c0:c0 + cb]
        HJ = hlp_rb[:, c0:c0 + cb]
        SJ = src_rs[:, c0:c0 + cb]
        DJ = dst_rs[:, c0:c0 + cb]

        rx = A1x - B1x
        ry = A1y - B1y
        b = D1x * D2x + D1y * D2y
        c = D1x * rx + D1y * ry
        f = D2x * rx + D2y * ry
        denom = jnp.maximum(AA * EE - b * b, 1e-12)
        rden = 1.0 / denom
        s = jnp.clip((b * f - c * EE) * rden, 0.0, 1.0)
        t = jnp.clip((b * s + f) * IE, 0.0, 1.0)
        s = jnp.clip((b * t - c) * IA, 0.0, 1.0)
        dx = rx + s * D1x - t * D2x
        dy = ry + s * D1y - t * D2y
        dist = jnp.sqrt(jnp.maximum(dx * dx + dy * dy, 1e-24))

        mdx = MIx - MJx
        mdy = MIy - MJy
        reach = HI + HJ
        prox = (mdx * mdx + mdy * mdy) < (reach * reach)
        share = ((SI == SJ) | (SI == DJ) | (DI == SJ) | (DI == DJ))
        cand = prox & jnp.logical_not(share)
        if tri_mask is not None:
            cand = cand & tri_mask
        candf = cand.astype(f32)
        contrib = candf * jnp.maximum(EPS - dist, 0.0)
        return jnp.sum(contrib), jnp.sum(candf)

    acc_loss = f32(0.0)
    acc_cnt = f32(0.0)
    nb_r = e // RB
    for bi in range(nb_r):
        r0 = bi * RB
        dl, dc_ = tile(r0, r0, DB, DB, tri)
        acc_loss = acc_loss + dl
        acc_cnt = acc_cnt + dc_
        for bj in range(bi + 1, e // CB):
            dl, dc_ = tile(r0, bj * CB, RB, CB, None)
            acc_loss = acc_loss + dl
            acc_cnt = acc_cnt + dc_

    out_ref[0, 0] = acc_loss / jnp.maximum(acc_cnt, 1.0)


def kernel(node_positions, adjacency, edge_index):
    del adjacency
    n = node_positions.shape[1]
    e = edge_index.shape[1]
    pos = node_positions.reshape(n, 2)
    posx = pos[:, 0]
    posy = pos[:, 1]
    src = edge_index[0]
    dst = edge_index[1]
    out = pl.pallas_call(
        _body,
        out_shape=jax.ShapeDtypeStruct((1, 1), jnp.float32),
        out_specs=pl.BlockSpec(memory_space=pltpu.SMEM),
    )(
        posx.reshape(1, n), posy.reshape(1, n),
        posx.reshape(n, 1), posy.reshape(n, 1),
        src.reshape(e, 1), dst.reshape(e, 1),
        src.reshape(1, e), dst.reshape(1, e),
    )
    return out[0, 0]
